# baseline (speedup 1.0000x reference)
import jax
import jax.numpy as jnp
from jax import lax
from jax.experimental import pallas as pl
from jax.experimental.pallas import tpu as pltpu
from jax.experimental.pallas import tpu_sc as plsc

_EPS = 1.0
_LAMB = 0.5
_N_NODES = 50000
_N_EDGES = 1600000

_NW = 32
_EA = 819200
_EB = _N_EDGES - _EA
_WA = _EA // _NW
_WB = _EB // _NW
_NODE_WORKERS = 25
_NCH = _N_NODES // _NODE_WORKERS


def _node_nll_partial(mu_b, s2_b, v0_b, v1_b, nvec, acc):
    def body(j, a):
        o = j * 16
        mu = mu_b[pl.ds(o, 16)]
        s2 = s2_b[pl.ds(o, 16)]
        v0 = v0_b[pl.ds(o, 16)]
        v1 = v1_b[pl.ds(o, 16)]
        x = s2 + _EPS
        t = s2 / (s2 + 2.0)
        t2 = t * t
        lg = t * (2.0 + t2 * (2.0 / 3.0 + t2 * (2.0 / 5.0 + t2 * (2.0 / 7.0 + t2 * (2.0 / 9.0)))))
        d = v0 - mu
        return a + (0.5 * lg + d * d / x) * v1

    return lax.fori_loop(0, nvec, body, acc)


def _gather_a_body(e_mu, e_s2, ekey, gemu, ges2,
                   key_b, mu_b, s2_b, semg, semw):
    cid = lax.axis_index("c")
    sid = lax.axis_index("s")
    wid = sid * 2 + cid
    eb = pl.multiple_of(wid * _WA, 8)
    pltpu.sync_copy(ekey.at[pl.ds(eb, _WA)], key_b)
    gm = pltpu.async_copy(e_mu.at[key_b], mu_b, semg)
    gs = pltpu.async_copy(e_s2.at[key_b], s2_b, semg)
    gm.wait()
    gs.wait()
    wm = pltpu.async_copy(mu_b, gemu.at[pl.ds(eb, _WA)], semw)
    ws = pltpu.async_copy(s2_b, ges2.at[pl.ds(eb, _WA)], semw)
    wm.wait()
    ws.wait()


_sc_gather_a = pl.kernel(
    _gather_a_body,
    out_type=(jax.ShapeDtypeStruct((_EA,), jnp.float32),
              jax.ShapeDtypeStruct((_EA,), jnp.float32)),
    mesh=plsc.VectorSubcoreMesh(core_axis_name="c", subcore_axis_name="s"),
    scratch_types=[
        pltpu.VMEM((_WA,), jnp.int32),
        pltpu.VMEM((_WA,), jnp.float32),
        pltpu.VMEM((_WA,), jnp.float32),
        pltpu.SemaphoreType.DMA,
        pltpu.SemaphoreType.DMA,
    ],
)


def _gather_b_body(n_mu, n_s2, e_mu, e_s2, nkey, nv0, nv1, ekey,
                   out_np, gemu, ges2,
                   key_b, mu_b, s2_b, nv0_b, nv1_b, stage_b, semg, semw):
    cid = lax.axis_index("c")
    sid = lax.axis_index("s")
    wid = sid * 2 + cid

    stage_b[...] = jnp.zeros((16,), jnp.float32)

    @pl.when(wid < _NODE_WORKERS)
    def _():
        nb = pl.multiple_of(wid * _NCH, 8)
        kv = key_b.at[pl.ds(0, _NCH)]
        pltpu.sync_copy(nkey.at[pl.ds(nb, _NCH)], kv)
        c0 = pltpu.async_copy(n_mu.at[kv], mu_b.at[pl.ds(0, _NCH)], semg)
        c1 = pltpu.async_copy(n_s2.at[kv], s2_b.at[pl.ds(0, _NCH)], semg)
        c2 = pltpu.async_copy(nv0.at[pl.ds(nb, _NCH)], nv0_b, semw)
        c3 = pltpu.async_copy(nv1.at[pl.ds(nb, _NCH)], nv1_b, semw)
        c0.wait()
        c1.wait()
        c2.wait()
        c3.wait()
        acc = _node_nll_partial(mu_b, s2_b, nv0_b, nv1_b, _NCH // 16,
                                jnp.zeros((16,), jnp.float32))
        stage_b[...] = acc

    pltpu.sync_copy(stage_b, out_np.at[wid])

    eb = pl.multiple_of(_EA + wid * _WB, 8)
    ob = pl.multiple_of(wid * _WB, 8)
    pltpu.sync_copy(ekey.at[pl.ds(eb, _WB)], key_b)
    gm = pltpu.async_copy(e_mu.at[key_b], mu_b, semg)
    gs = pltpu.async_copy(e_s2.at[key_b], s2_b, semg)
    gm.wait()
    gs.wait()
    wm = pltpu.async_copy(mu_b, gemu.at[pl.ds(ob, _WB)], semw)
    ws = pltpu.async_copy(s2_b, ges2.at[pl.ds(ob, _WB)], semw)
    wm.wait()
    ws.wait()


_sc_gather_b = pl.kernel(
    _gather_b_body,
    out_type=(jax.ShapeDtypeStruct((_NW, 16), jnp.float32),
              jax.ShapeDtypeStruct((_EB,), jnp.float32),
              jax.ShapeDtypeStruct((_EB,), jnp.float32)),
    mesh=plsc.VectorSubcoreMesh(core_axis_name="c", subcore_axis_name="s"),
    scratch_types=[
        pltpu.VMEM((_WB,), jnp.int32),
        pltpu.VMEM((_WB,), jnp.float32),
        pltpu.VMEM((_WB,), jnp.float32),
        pltpu.VMEM((_NCH,), jnp.float32),
        pltpu.VMEM((_NCH,), jnp.float32),
        pltpu.VMEM((16,), jnp.float32),
        pltpu.SemaphoreType.DMA,
        pltpu.SemaphoreType.DMA,
    ],
)

_BR = 800
_GA = _EA // 128 // _BR
_GB = 8
_BROWS = _EB // 128
_TAILR = _BROWS - (_GB - 1) * _BR


def _nll_tc_a_body(gemu, ges2, ev0, ev1, oa_ref, acce):
    pid = pl.program_id(0)

    @pl.when(pid == 0)
    def _():
        acce[0] = 0.0

    x = ges2[...] + _EPS
    d = ev0[...] - gemu[...]
    acce[0] += jnp.sum((0.5 * jnp.log(x) + d * d / x) * ev1[...])

    @pl.when(pid == _GA - 1)
    def _():
        oa_ref[0, 0] = acce[0]


_nll_tc_a = pl.pallas_call(
    _nll_tc_a_body,
    grid=(_GA,),
    in_specs=[
        pl.BlockSpec((_BR, 128), lambda i: (i, 0)),
        pl.BlockSpec((_BR, 128), lambda i: (i, 0)),
        pl.BlockSpec((_BR, 128), lambda i: (i, 0)),
        pl.BlockSpec((_BR, 128), lambda i: (i, 0)),
    ],
    out_shape=jax.ShapeDtypeStruct((1, 1), jnp.float32),
    out_specs=pl.BlockSpec(memory_space=pltpu.SMEM),
    scratch_shapes=[pltpu.SMEM((1,), jnp.float32)],
)


def _nll_tc_b_body(np_ref, oa_ref, gemu, ges2, ev0, ev1,
                   on_ref, oe_ref, ot_ref, acce):
    pid = pl.program_id(0)

    @pl.when(pid == 0)
    def _():
        acce[0] = 0.0

    x = ges2[...] + _EPS
    d = ev0[...] - gemu[...]
    term = (0.5 * jnp.log(x) + d * d / x) * ev1[...]

    @pl.when(pid < _GB - 1)
    def _():
        acce[0] += jnp.sum(term)

    @pl.when(pid == _GB - 1)
    def _():
        rows = lax.broadcasted_iota(jnp.int32, (_BR, 128), 0)
        acce[0] += jnp.sum(jnp.where(rows < _TAILR, term, 0.0))
        e = (acce[0] + oa_ref[0, 0]) * (1.0 / _N_EDGES)
        n = jnp.sum(np_ref[...]) * (1.0 / _N_NODES)
        on_ref[0, 0] = n
        oe_ref[0, 0] = e
        ot_ref[0, 0] = n * _LAMB + e * (1.0 - _LAMB)


_nll_tc_b = pl.pallas_call(
    _nll_tc_b_body,
    grid=(_GB,),
    in_specs=[
        pl.BlockSpec((_NW, 16), lambda i: (0, 0)),
        pl.BlockSpec(memory_space=pltpu.SMEM),
        pl.BlockSpec((_BR, 128), lambda i: (i, 0)),
        pl.BlockSpec((_BR, 128), lambda i: (i, 0)),
        pl.BlockSpec((_BR, 128), lambda i: (i + _GA, 0)),
        pl.BlockSpec((_BR, 128), lambda i: (i + _GA, 0)),
    ],
    out_shape=(jax.ShapeDtypeStruct((1, 1), jnp.float32),
               jax.ShapeDtypeStruct((1, 1), jnp.float32),
               jax.ShapeDtypeStruct((1, 1), jnp.float32)),
    out_specs=(pl.BlockSpec(memory_space=pltpu.SMEM),
               pl.BlockSpec(memory_space=pltpu.SMEM),
               pl.BlockSpec(memory_space=pltpu.SMEM)),
    scratch_shapes=[pltpu.SMEM((1,), jnp.float32)],
)


def kernel(n_mu, n_sigma2, e_mu, e_sigma2, batch_node_key, batch_node_value,
           batch_edge_key, batch_edge_value):
    ekey = batch_edge_key.astype(jnp.int32)
    ev0 = batch_edge_value[:, 0].reshape(_N_EDGES // 128, 128)
    ev1 = batch_edge_value[:, 1].reshape(_N_EDGES // 128, 128)
    gemu_a, ges2_a = _sc_gather_a(e_mu, e_sigma2, ekey)
    acc_a = _nll_tc_a(gemu_a.reshape(_EA // 128, 128),
                      ges2_a.reshape(_EA // 128, 128), ev0, ev1)
    node_pp, gemu_b, ges2_b = _sc_gather_b(
        n_mu, n_sigma2, e_mu, e_sigma2,
        batch_node_key.astype(jnp.int32),
        batch_node_value[:, 0], batch_node_value[:, 1], ekey)
    on, oe, ot = _nll_tc_b(
        node_pp, acc_a,
        gemu_b.reshape(_BROWS, 128), ges2_b.reshape(_BROWS, 128),
        ev0, ev1)
    return (on[0, 0], oe[0, 0], ot[0, 0])


# speedup vs baseline: 1.0248x; 1.0248x over previous
import jax
import jax.numpy as jnp
from jax import lax
from jax.experimental import pallas as pl
from jax.experimental.pallas import tpu as pltpu
from jax.experimental.pallas import tpu_sc as plsc

_EPS = 1.0
_LAMB = 0.5
_N_NODES = 50000
_N_EDGES = 1600000

_NW = 32
_E_PER_W = _N_EDGES // _NW
_GCH = 25000
_NODE_WORKERS = 25
_NCH = _N_NODES // _NODE_WORKERS


def _node_nll_partial(mu_b, s2_b, v0_b, v1_b, nvec, acc):
    def body(j, a):
        o = j * 16
        mu = mu_b[pl.ds(o, 16)]
        s2 = s2_b[pl.ds(o, 16)]
        v0 = v0_b[pl.ds(o, 16)]
        v1 = v1_b[pl.ds(o, 16)]
        x = s2 + _EPS
        t = s2 / (s2 + 2.0)
        t2 = t * t
        lg = t * (2.0 + t2 * (2.0 / 3.0 + t2 * (2.0 / 5.0 + t2 * (2.0 / 7.0 + t2 * (2.0 / 9.0)))))
        d = v0 - mu
        return a + (0.5 * lg + d * d / x) * v1

    return lax.fori_loop(0, nvec, body, acc)


def _gather_body(n_mu, n_s2, e_mu, e_s2, nkey, nv0, nv1, ekey,
                 out_np, gemu, ges2,
                 key_b, mu0_b, s20_b, mu1_b, s21_b, stage_b, semg, semw):
    cid = lax.axis_index("c")
    sid = lax.axis_index("s")
    wid = sid * 2 + cid

    stage_b[...] = jnp.zeros((16,), jnp.float32)

    @pl.when(wid < _NODE_WORKERS)
    def _():
        nb = pl.multiple_of(wid * _NCH, 8)
        kv = key_b.at[pl.ds(0, _NCH)]
        pltpu.sync_copy(nkey.at[pl.ds(nb, _NCH)], kv)
        c0 = pltpu.async_copy(n_mu.at[kv], mu0_b.at[pl.ds(0, _NCH)], semg)
        c1 = pltpu.async_copy(n_s2.at[kv], s20_b.at[pl.ds(0, _NCH)], semg)
        c2 = pltpu.async_copy(nv0.at[pl.ds(nb, _NCH)], mu1_b.at[pl.ds(0, _NCH)], semw)
        c3 = pltpu.async_copy(nv1.at[pl.ds(nb, _NCH)], s21_b.at[pl.ds(0, _NCH)], semw)
        c0.wait()
        c1.wait()
        c2.wait()
        c3.wait()
        acc = _node_nll_partial(mu0_b, s20_b, mu1_b, s21_b, _NCH // 16,
                                jnp.zeros((16,), jnp.float32))
        stage_b[...] = acc

    pltpu.sync_copy(stage_b, out_np.at[wid])

    eb0 = pl.multiple_of(wid * _E_PER_W, 8)
    eb1 = pl.multiple_of(wid * _E_PER_W + _GCH, 8)

    pltpu.sync_copy(ekey.at[pl.ds(eb0, _GCH)], key_b)
    g0m = pltpu.async_copy(e_mu.at[key_b], mu0_b, semg)
    g0s = pltpu.async_copy(e_s2.at[key_b], s20_b, semg)
    g0m.wait()
    g0s.wait()
    w0m = pltpu.async_copy(mu0_b, gemu.at[pl.ds(eb0, _GCH)], semw)
    w0s = pltpu.async_copy(s20_b, ges2.at[pl.ds(eb0, _GCH)], semw)

    pltpu.sync_copy(ekey.at[pl.ds(eb1, _GCH)], key_b)
    g1m = pltpu.async_copy(e_mu.at[key_b], mu1_b, semg)
    g1s = pltpu.async_copy(e_s2.at[key_b], s21_b, semg)
    g1m.wait()
    g1s.wait()
    w1m = pltpu.async_copy(mu1_b, gemu.at[pl.ds(eb1, _GCH)], semw)
    w1s = pltpu.async_copy(s21_b, ges2.at[pl.ds(eb1, _GCH)], semw)

    w0m.wait()
    w0s.wait()
    w1m.wait()
    w1s.wait()


_sc_gather = pl.kernel(
    _gather_body,
    out_type=(jax.ShapeDtypeStruct((_NW, 16), jnp.float32),
              jax.ShapeDtypeStruct((_N_EDGES,), jnp.float32),
              jax.ShapeDtypeStruct((_N_EDGES,), jnp.float32)),
    mesh=plsc.VectorSubcoreMesh(core_axis_name="c", subcore_axis_name="s"),
    scratch_types=[
        pltpu.VMEM((_GCH,), jnp.int32),
        pltpu.VMEM((_GCH,), jnp.float32),
        pltpu.VMEM((_GCH,), jnp.float32),
        pltpu.VMEM((_GCH,), jnp.float32),
        pltpu.VMEM((_GCH,), jnp.float32),
        pltpu.VMEM((16,), jnp.float32),
        pltpu.SemaphoreType.DMA,
        pltpu.SemaphoreType.DMA,
    ],
)

_EROWS = 12500
_BR = 2048
_GE = -(-_EROWS // _BR)
_TAILR = _EROWS - (_GE - 1) * _BR


def _nll_tc_body(np_ref, gemu, ges2, ev0, ev1,
                 on_ref, oe_ref, ot_ref, acce):
    pid = pl.program_id(0)

    @pl.when(pid == 0)
    def _():
        acce[0] = 0.0

    x = ges2[...] + _EPS
    d = ev0[...] - gemu[...]
    term = (0.5 * jnp.log(x) + d * d / x) * ev1[...]

    @pl.when(pid < _GE - 1)
    def _():
        acce[0] += jnp.sum(term)

    @pl.when(pid == _GE - 1)
    def _():
        rows = lax.broadcasted_iota(jnp.int32, (_BR, 128), 0)
        acce[0] += jnp.sum(jnp.where(rows < _TAILR, term, 0.0))
        e = acce[0] * (1.0 / _N_EDGES)
        n = jnp.sum(np_ref[...]) * (1.0 / _N_NODES)
        on_ref[0, 0] = n
        oe_ref[0, 0] = e
        ot_ref[0, 0] = n * _LAMB + e * (1.0 - _LAMB)


_nll_tc = pl.pallas_call(
    _nll_tc_body,
    grid=(_GE,),
    in_specs=[
        pl.BlockSpec((_NW, 16), lambda i: (0, 0)),
        pl.BlockSpec((_BR, 128), lambda i: (i, 0)),
        pl.BlockSpec((_BR, 128), lambda i: (i, 0)),
        pl.BlockSpec((_BR, 128), lambda i: (i, 0)),
        pl.BlockSpec((_BR, 128), lambda i: (i, 0)),
    ],
    out_shape=(jax.ShapeDtypeStruct((1, 1), jnp.float32),
               jax.ShapeDtypeStruct((1, 1), jnp.float32),
               jax.ShapeDtypeStruct((1, 1), jnp.float32)),
    out_specs=(pl.BlockSpec(memory_space=pltpu.SMEM),
               pl.BlockSpec(memory_space=pltpu.SMEM),
               pl.BlockSpec(memory_space=pltpu.SMEM)),
    scratch_shapes=[pltpu.SMEM((1,), jnp.float32)],
)


def kernel(n_mu, n_sigma2, e_mu, e_sigma2, batch_node_key, batch_node_value,
           batch_edge_key, batch_edge_value):
    node_pp, gemu, ges2 = _sc_gather(
        n_mu, n_sigma2, e_mu, e_sigma2,
        batch_node_key.astype(jnp.int32),
        batch_node_value[:, 0], batch_node_value[:, 1],
        batch_edge_key.astype(jnp.int32))
    on, oe, ot = _nll_tc(
        node_pp, gemu.reshape(_EROWS, 128), ges2.reshape(_EROWS, 128),
        batch_edge_value[:, 0].reshape(_EROWS, 128),
        batch_edge_value[:, 1].reshape(_EROWS, 128))
    return (on[0, 0], oe[0, 0], ot[0, 0])
